# Initial kernel scaffold; baseline (speedup 1.0000x reference)
import functools

import jax
import jax.numpy as jnp
from jax import lax
from jax.experimental import pallas as pl
from jax.experimental.pallas import tpu as pltpu
from jax.experimental.pallas import tpu_sc as plsc

N = 50000
E = 800000
H = 64
HH = H // 2
NC = 2
NS = 16
CHUNK = 128

EPT_L = E // NS
NFULL_L = EPT_L // CHUNK
TAIL_L = EPT_L - NFULL_L * CHUNK

EPT_D = E // (NC * NS)
NFULL_D = EPT_D // CHUNK
TAIL_D = EPT_D - NFULL_D * CHUNK

_mesh = plsc.VectorSubcoreMesh(core_axis_name="c", subcore_axis_name="s")


def _leaky(v):
    return jnp.where(v >= 0, v, 0.01 * v)


_BLK = 1000
_GRID = N // _BLK


def _embed_body(x_ref, we0_ref, be0_ref, we1_ref, be1_ref, out_ref):
    h = _leaky(x_ref[...] * we0_ref[...] + be0_ref[...])
    h = jnp.dot(h, we1_ref[...], preferred_element_type=jnp.float32)
    out_ref[...] = _leaky(h + be1_ref[...])


def _embed(x, we0, be0, we1, be1):
    return pl.pallas_call(
        _embed_body,
        grid=(_GRID,),
        in_specs=[
            pl.BlockSpec((_BLK, 1), lambda i: (i, 0)),
            pl.BlockSpec((1, H), lambda i: (0, 0)),
            pl.BlockSpec((1, H), lambda i: (0, 0)),
            pl.BlockSpec((H, H), lambda i: (0, 0)),
            pl.BlockSpec((1, H), lambda i: (0, 0)),
        ],
        out_specs=pl.BlockSpec((_BLK, H), lambda i: (i, 0)),
        out_shape=jax.ShapeDtypeStruct((N, H), jnp.float32),
    )(x, we0, be0, we1, be1)


def _layer_body(lo_ref, hi_ref, d0_ref, d1_ref, h_ref, wl_ref, bl_ref,
                wr_ref, out_ref, *, residual):
    deg = jnp.maximum(d0_ref[:, 0:1] + d1_ref[:, 0:1], 1.0)
    invd = 1.0 / deg
    lo = lo_ref[...] * invd
    hi = hi_ref[...] * invd
    h = h_ref[...]
    acc = (jnp.dot(lo, wl_ref[0:HH, :], preferred_element_type=jnp.float32)
           + jnp.dot(hi, wl_ref[HH:H, :], preferred_element_type=jnp.float32)
           + jnp.dot(h, wr_ref[...], preferred_element_type=jnp.float32)
           + bl_ref[...])
    act = _leaky(acc)
    out_ref[...] = act + h if residual else act


def _layer(lo, hi, d0, d1, h, wl, bl, wr, residual):
    return pl.pallas_call(
        functools.partial(_layer_body, residual=residual),
        grid=(_GRID,),
        in_specs=[
            pl.BlockSpec((_BLK, HH), lambda i: (i, 0)),
            pl.BlockSpec((_BLK, HH), lambda i: (i, 0)),
            pl.BlockSpec((_BLK, 8), lambda i: (i, 0)),
            pl.BlockSpec((_BLK, 8), lambda i: (i, 0)),
            pl.BlockSpec((_BLK, H), lambda i: (i, 0)),
            pl.BlockSpec((H, H), lambda i: (0, 0)),
            pl.BlockSpec((1, H), lambda i: (0, 0)),
            pl.BlockSpec((H, H), lambda i: (0, 0)),
        ],
        out_specs=pl.BlockSpec((_BLK, H), lambda i: (i, 0)),
        out_shape=jax.ShapeDtypeStruct((N, H), jnp.float32),
    )(lo, hi, d0, d1, h, wl, bl, wr)


def _outmlp_body(h_ref, wo0_ref, bo0_ref, wo1_ref, bo1_ref, out_ref):
    t = _leaky(jnp.dot(h_ref[...], wo0_ref[...],
                       preferred_element_type=jnp.float32) + bo0_ref[...])
    out_ref[...] = jnp.dot(t, wo1_ref[...],
                           preferred_element_type=jnp.float32) + bo1_ref[...]


def _outmlp(h, wo0, bo0, wo1, bo1):
    return pl.pallas_call(
        _outmlp_body,
        grid=(_GRID,),
        in_specs=[
            pl.BlockSpec((_BLK, H), lambda i: (i, 0)),
            pl.BlockSpec((H, H), lambda i: (0, 0)),
            pl.BlockSpec((1, H), lambda i: (0, 0)),
            pl.BlockSpec((H, H), lambda i: (0, 0)),
            pl.BlockSpec((1, H), lambda i: (0, 0)),
        ],
        out_specs=pl.BlockSpec((_BLK, H), lambda i: (i, 0)),
        out_shape=jax.ShapeDtypeStruct((N, H), jnp.float32),
    )(h, wo0, bo0, wo1, bo1)


@functools.partial(
    pl.kernel,
    out_type=(jax.ShapeDtypeStruct((N, 8), jnp.float32),
              jax.ShapeDtypeStruct((N, 8), jnp.float32)),
    mesh=_mesh,
    scratch_types=[
        pltpu.VMEM((CHUNK, 8), jnp.float32),
        pltpu.VMEM((CHUNK,), jnp.int32),
        pltpu.VMEM((TAIL_D,), jnp.int32),
        pltpu.VMEM_SHARED((N, 8), jnp.float32),
    ],
)
def _deg_kernel(dst_hbm, zeros8_hbm, ones_hbm, out0, out1,
                ones_v, dst_v, dst_t, acc):
    c = lax.axis_index("c")
    s = lax.axis_index("s")

    @pl.when(s == 0)
    def _():
        pltpu.sync_copy(zeros8_hbm, acc)

    pltpu.sync_copy(ones_hbm, ones_v)
    plsc.subcore_barrier()

    base = (c * NS + s) * EPT_D

    def body(i, _):
        e0 = pl.multiple_of(base + i * CHUNK, 8)
        pltpu.sync_copy(dst_hbm.at[pl.ds(e0, CHUNK)], dst_v)
        pltpu.sync_copy(ones_v, acc.at[dst_v], add=True)
        return 0

    lax.fori_loop(0, NFULL_D, body, 0)
    e0 = pl.multiple_of(base + NFULL_D * CHUNK, 8)
    pltpu.sync_copy(dst_hbm.at[pl.ds(e0, TAIL_D)], dst_t)
    pltpu.sync_copy(ones_v.at[pl.ds(0, TAIL_D)], acc.at[dst_t], add=True)

    plsc.subcore_barrier()

    @pl.when(s == 0)
    def _():
        @pl.when(c == 0)
        def _():
            pltpu.sync_copy(acc, out0)

        @pl.when(c == 1)
        def _():
            pltpu.sync_copy(acc, out1)


@functools.partial(
    pl.kernel,
    out_type=(jax.ShapeDtypeStruct((N, HH), jnp.float32),
              jax.ShapeDtypeStruct((N, HH), jnp.float32)),
    mesh=_mesh,
    scratch_types=[
        pltpu.VMEM((CHUNK,), jnp.int32),
        pltpu.VMEM((CHUNK,), jnp.int32),
        pltpu.VMEM((CHUNK,), jnp.int32),
        pltpu.VMEM((CHUNK, HH), jnp.float32),
        pltpu.VMEM((TAIL_L,), jnp.int32),
        pltpu.VMEM((TAIL_L,), jnp.int32),
        pltpu.VMEM((TAIL_L,), jnp.int32),
        pltpu.VMEM((TAIL_L, HH), jnp.float32),
        pltpu.VMEM_SHARED((N, HH), jnp.float32),
    ],
)
def _seg_kernel(ht_hbm, src_hbm, dst_hbm, zeros32_hbm, out_lo, out_hi,
                src_v, idx_v, dst_v, rows_v, src_t, idx_t, dst_t, rows_t, acc):
    c = lax.axis_index("c")
    s = lax.axis_index("s")

    @pl.when(s == 0)
    def _():
        pltpu.sync_copy(zeros32_hbm, acc)

    plsc.subcore_barrier()

    base = s * EPT_L

    def do_chunk(e0, nb, src_ref, idx_ref, dst_ref, rows_ref):
        pltpu.sync_copy(src_hbm.at[pl.ds(e0, nb * 16)], src_ref)
        pltpu.sync_copy(dst_hbm.at[pl.ds(e0, nb * 16)], dst_ref)
        for j in range(nb):
            sv = src_ref[pl.ds(j * 16, 16)]
            idx_ref[pl.ds(j * 16, 16)] = sv * 2 + c
        pltpu.sync_copy(ht_hbm.at[idx_ref], rows_ref)
        pltpu.sync_copy(rows_ref, acc.at[dst_ref], add=True)

    def body(i, _):
        e0 = pl.multiple_of(base + i * CHUNK, 8)
        do_chunk(e0, CHUNK // 16, src_v, idx_v, dst_v, rows_v)
        return 0

    lax.fori_loop(0, NFULL_L, body, 0)
    e0 = pl.multiple_of(base + NFULL_L * CHUNK, 8)
    do_chunk(e0, TAIL_L // 16, src_t, idx_t, dst_t, rows_t)

    plsc.subcore_barrier()

    @pl.when(s == 0)
    def _():
        @pl.when(c == 0)
        def _():
            pltpu.sync_copy(acc, out_lo)

        @pl.when(c == 1)
        def _():
            pltpu.sync_copy(acc, out_hi)


def kernel(x, edge_index, W_e0, b_e0, W_e1, b_e1, Wl, bl, Wr,
           W_o0, b_o0, W_o1, b_o1):
    src = edge_index[0]
    dst = edge_index[1]

    zeros8 = jnp.zeros((N, 8), jnp.float32)
    zeros32 = jnp.zeros((N, HH), jnp.float32)
    ones = jnp.ones((CHUNK, 8), jnp.float32)

    h = _embed(x, W_e0, b_e0.reshape(1, H), W_e1, b_e1.reshape(1, H))
    d0, d1 = _deg_kernel(dst, zeros8, ones)

    for i in range(3):
        lo, hi = _seg_kernel(h.reshape(2 * N, HH), src, dst, zeros32)
        h = _layer(lo, hi, d0, d1, h, Wl[i], bl[i].reshape(1, H), Wr[i],
                   residual=(i == 0))

    return _outmlp(h, W_o0, b_o0.reshape(1, H), W_o1, b_o1.reshape(1, H))


# speedup vs baseline: 3.6603x; 3.6603x over previous
import functools

import jax
import jax.numpy as jnp
from jax import lax
from jax.experimental import pallas as pl
from jax.experimental.pallas import tpu as pltpu
from jax.experimental.pallas import tpu_sc as plsc

N = 50000
E = 800000
H = 64
HH = H // 2
NC = 2
NS = 16
CHUNK = 128

EPT_L = E // NS
NFULL_L = EPT_L // CHUNK
TAIL_L = EPT_L - NFULL_L * CHUNK

EPT_D = E // (NC * NS)
NFULL_D = EPT_D // CHUNK
TAIL_D = EPT_D - NFULL_D * CHUNK

_mesh = plsc.VectorSubcoreMesh(core_axis_name="c", subcore_axis_name="s")


def _leaky(v):
    return jnp.where(v >= 0, v, 0.01 * v)


_BLK = 1000
_GRID = N // _BLK


def _embed_body(x_ref, we0_ref, be0_ref, we1_ref, be1_ref, out_ref):
    h = _leaky(x_ref[...] * we0_ref[...] + be0_ref[...])
    h = jnp.dot(h, we1_ref[...], preferred_element_type=jnp.float32)
    out_ref[...] = _leaky(h + be1_ref[...])


def _embed(x, we0, be0, we1, be1):
    return pl.pallas_call(
        _embed_body,
        grid=(_GRID,),
        in_specs=[
            pl.BlockSpec((_BLK, 1), lambda i: (i, 0)),
            pl.BlockSpec((1, H), lambda i: (0, 0)),
            pl.BlockSpec((1, H), lambda i: (0, 0)),
            pl.BlockSpec((H, H), lambda i: (0, 0)),
            pl.BlockSpec((1, H), lambda i: (0, 0)),
        ],
        out_specs=pl.BlockSpec((_BLK, H), lambda i: (i, 0)),
        out_shape=jax.ShapeDtypeStruct((N, H), jnp.float32),
    )(x, we0, be0, we1, be1)


def _layer_body(lo_ref, hi_ref, d0_ref, d1_ref, h_ref, wl_ref, bl_ref,
                wr_ref, out_ref, *, residual):
    deg = jnp.maximum(d0_ref[:, 0:1] + d1_ref[:, 0:1], 1.0)
    invd = 1.0 / deg
    lo = lo_ref[...] * invd
    hi = hi_ref[...] * invd
    h = h_ref[...]
    acc = (jnp.dot(lo, wl_ref[0:HH, :], preferred_element_type=jnp.float32)
           + jnp.dot(hi, wl_ref[HH:H, :], preferred_element_type=jnp.float32)
           + jnp.dot(h, wr_ref[...], preferred_element_type=jnp.float32)
           + bl_ref[...])
    act = _leaky(acc)
    out_ref[...] = act + h if residual else act


def _layer(lo, hi, d0, d1, h, wl, bl, wr, residual):
    return pl.pallas_call(
        functools.partial(_layer_body, residual=residual),
        grid=(_GRID,),
        in_specs=[
            pl.BlockSpec((_BLK, HH), lambda i: (i, 0)),
            pl.BlockSpec((_BLK, HH), lambda i: (i, 0)),
            pl.BlockSpec((_BLK, 8), lambda i: (i, 0)),
            pl.BlockSpec((_BLK, 8), lambda i: (i, 0)),
            pl.BlockSpec((_BLK, H), lambda i: (i, 0)),
            pl.BlockSpec((H, H), lambda i: (0, 0)),
            pl.BlockSpec((1, H), lambda i: (0, 0)),
            pl.BlockSpec((H, H), lambda i: (0, 0)),
        ],
        out_specs=pl.BlockSpec((_BLK, H), lambda i: (i, 0)),
        out_shape=jax.ShapeDtypeStruct((N, H), jnp.float32),
    )(lo, hi, d0, d1, h, wl, bl, wr)


def _outmlp_body(h_ref, wo0_ref, bo0_ref, wo1_ref, bo1_ref, out_ref):
    t = _leaky(jnp.dot(h_ref[...], wo0_ref[...],
                       preferred_element_type=jnp.float32) + bo0_ref[...])
    out_ref[...] = jnp.dot(t, wo1_ref[...],
                           preferred_element_type=jnp.float32) + bo1_ref[...]


def _outmlp(h, wo0, bo0, wo1, bo1):
    return pl.pallas_call(
        _outmlp_body,
        grid=(_GRID,),
        in_specs=[
            pl.BlockSpec((_BLK, H), lambda i: (i, 0)),
            pl.BlockSpec((H, H), lambda i: (0, 0)),
            pl.BlockSpec((1, H), lambda i: (0, 0)),
            pl.BlockSpec((H, H), lambda i: (0, 0)),
            pl.BlockSpec((1, H), lambda i: (0, 0)),
        ],
        out_specs=pl.BlockSpec((_BLK, H), lambda i: (i, 0)),
        out_shape=jax.ShapeDtypeStruct((N, H), jnp.float32),
    )(h, wo0, bo0, wo1, bo1)


@functools.partial(
    pl.kernel,
    out_type=(jax.ShapeDtypeStruct((N, 8), jnp.float32),
              jax.ShapeDtypeStruct((N, 8), jnp.float32)),
    mesh=_mesh,
    scratch_types=[
        pltpu.VMEM((CHUNK, 8), jnp.float32),
        pltpu.VMEM((CHUNK,), jnp.int32),
        pltpu.VMEM((TAIL_D,), jnp.int32),
        pltpu.VMEM_SHARED((N, 8), jnp.float32),
    ],
    compiler_params=pltpu.CompilerParams(use_tc_tiling_on_sc=False),
)
def _deg_kernel(dst_hbm, zeros8_hbm, ones_hbm, out0, out1,
                ones_v, dst_v, dst_t, acc):
    c = lax.axis_index("c")
    s = lax.axis_index("s")

    @pl.when(s == 0)
    def _():
        pltpu.sync_copy(zeros8_hbm, acc)

    pltpu.sync_copy(ones_hbm, ones_v)
    plsc.subcore_barrier()

    base = (c * NS + s) * EPT_D

    def body(i, _):
        e0 = pl.multiple_of(base + i * CHUNK, 8)
        pltpu.sync_copy(dst_hbm.at[pl.ds(e0, CHUNK)], dst_v)
        pltpu.sync_copy(ones_v, acc.at[dst_v], add=True)
        return 0

    lax.fori_loop(0, NFULL_D, body, 0)
    e0 = pl.multiple_of(base + NFULL_D * CHUNK, 8)
    pltpu.sync_copy(dst_hbm.at[pl.ds(e0, TAIL_D)], dst_t)
    pltpu.sync_copy(ones_v.at[pl.ds(0, TAIL_D)], acc.at[dst_t], add=True)

    plsc.subcore_barrier()

    @pl.when(s == 0)
    def _():
        @pl.when(c == 0)
        def _():
            pltpu.sync_copy(acc, out0)

        @pl.when(c == 1)
        def _():
            pltpu.sync_copy(acc, out1)


@functools.partial(
    pl.kernel,
    out_type=(jax.ShapeDtypeStruct((N, HH), jnp.float32),
              jax.ShapeDtypeStruct((N, HH), jnp.float32)),
    mesh=_mesh,
    scratch_types=[
        pltpu.VMEM((CHUNK,), jnp.int32),
        pltpu.VMEM((CHUNK,), jnp.int32),
        pltpu.VMEM((CHUNK,), jnp.int32),
        pltpu.VMEM((CHUNK, HH), jnp.float32),
        pltpu.VMEM((TAIL_L,), jnp.int32),
        pltpu.VMEM((TAIL_L,), jnp.int32),
        pltpu.VMEM((TAIL_L,), jnp.int32),
        pltpu.VMEM((TAIL_L, HH), jnp.float32),
        pltpu.VMEM_SHARED((N, HH), jnp.float32),
    ],
    compiler_params=pltpu.CompilerParams(use_tc_tiling_on_sc=False),
)
def _seg_kernel(ht_hbm, src_hbm, dst_hbm, zeros32_hbm, out_lo, out_hi,
                src_v, idx_v, dst_v, rows_v, src_t, idx_t, dst_t, rows_t, acc):
    c = lax.axis_index("c")
    s = lax.axis_index("s")

    @pl.when(s == 0)
    def _():
        pltpu.sync_copy(zeros32_hbm, acc)

    plsc.subcore_barrier()

    base = s * EPT_L

    def do_chunk(e0, nb, src_ref, idx_ref, dst_ref, rows_ref):
        pltpu.sync_copy(src_hbm.at[pl.ds(e0, nb * 16)], src_ref)
        pltpu.sync_copy(dst_hbm.at[pl.ds(e0, nb * 16)], dst_ref)
        for j in range(nb):
            sv = src_ref[pl.ds(j * 16, 16)]
            idx_ref[pl.ds(j * 16, 16)] = sv * 2 + c
        pltpu.sync_copy(ht_hbm.at[idx_ref], rows_ref)
        pltpu.sync_copy(rows_ref, acc.at[dst_ref], add=True)

    def body(i, _):
        e0 = pl.multiple_of(base + i * CHUNK, 8)
        do_chunk(e0, CHUNK // 16, src_v, idx_v, dst_v, rows_v)
        return 0

    lax.fori_loop(0, NFULL_L, body, 0)
    e0 = pl.multiple_of(base + NFULL_L * CHUNK, 8)
    do_chunk(e0, TAIL_L // 16, src_t, idx_t, dst_t, rows_t)

    plsc.subcore_barrier()

    @pl.when(s == 0)
    def _():
        @pl.when(c == 0)
        def _():
            pltpu.sync_copy(acc, out_lo)

        @pl.when(c == 1)
        def _():
            pltpu.sync_copy(acc, out_hi)


def kernel(x, edge_index, W_e0, b_e0, W_e1, b_e1, Wl, bl, Wr,
           W_o0, b_o0, W_o1, b_o1):
    src = edge_index[0]
    dst = edge_index[1]

    zeros8 = jnp.zeros((N, 8), jnp.float32)
    zeros32 = jnp.zeros((N, HH), jnp.float32)
    ones = jnp.ones((CHUNK, 8), jnp.float32)

    h = _embed(x, W_e0, b_e0.reshape(1, H), W_e1, b_e1.reshape(1, H))
    d0, d1 = _deg_kernel(dst, zeros8, ones)

    for i in range(3):
        lo, hi = _seg_kernel(h.reshape(2 * N, HH), src, dst, zeros32)
        h = _layer(lo, hi, d0, d1, h, Wl[i], bl[i].reshape(1, H), Wr[i],
                   residual=(i == 0))

    return _outmlp(h, W_o0, b_o0.reshape(1, H), W_o1, b_o1.reshape(1, H))


# speedup vs baseline: 7.9017x; 2.1588x over previous
import functools

import jax
import jax.numpy as jnp
from jax import lax
from jax.experimental import pallas as pl
from jax.experimental.pallas import tpu as pltpu
from jax.experimental.pallas import tpu_sc as plsc

N = 50000
E = 800000
H = 64
HH = H // 2
NC = 2
NS = 16
CHUNK = 128

G = 5
GE = G * CHUNK

EPT_L = E // NS
NGRP_L = EPT_L // GE
TAIL_L = EPT_L - NGRP_L * GE

EPT_D = E // (NC * NS)
NGRP_D = EPT_D // GE
TAIL_D = EPT_D - NGRP_D * GE

ECHUNKS = E // CHUNK

_mesh = plsc.VectorSubcoreMesh(core_axis_name="c", subcore_axis_name="s")


def _leaky(v):
    return jnp.where(v >= 0, v, 0.01 * v)


_BLK = 1000
_GRID = N // _BLK


def _embed_body(x_ref, we0_ref, be0_ref, we1_ref, be1_ref, out_ref):
    h = _leaky(x_ref[...] * we0_ref[...] + be0_ref[...])
    h = jnp.dot(h, we1_ref[...], preferred_element_type=jnp.float32)
    out_ref[...] = _leaky(h + be1_ref[...])


def _embed(x, we0, be0, we1, be1):
    return pl.pallas_call(
        _embed_body,
        grid=(_GRID,),
        in_specs=[
            pl.BlockSpec((_BLK, 1), lambda i: (i, 0)),
            pl.BlockSpec((1, H), lambda i: (0, 0)),
            pl.BlockSpec((1, H), lambda i: (0, 0)),
            pl.BlockSpec((H, H), lambda i: (0, 0)),
            pl.BlockSpec((1, H), lambda i: (0, 0)),
        ],
        out_specs=pl.BlockSpec((_BLK, H), lambda i: (i, 0)),
        out_shape=jax.ShapeDtypeStruct((N, H), jnp.float32),
    )(x, we0, be0, we1, be1)


def _layer_body(lo_ref, hi_ref, d0_ref, d1_ref, h_ref, wl_ref, bl_ref,
                wr_ref, out_ref, *, residual):
    deg = jnp.maximum(d0_ref[:, 0:1] + d1_ref[:, 0:1], 1.0)
    invd = 1.0 / deg
    lo = lo_ref[...] * invd
    hi = hi_ref[...] * invd
    h = h_ref[...]
    acc = (jnp.dot(lo, wl_ref[0:HH, :], preferred_element_type=jnp.float32)
           + jnp.dot(hi, wl_ref[HH:H, :], preferred_element_type=jnp.float32)
           + jnp.dot(h, wr_ref[...], preferred_element_type=jnp.float32)
           + bl_ref[...])
    act = _leaky(acc)
    out_ref[...] = act + h if residual else act


def _layer(lo, hi, d0, d1, h, wl, bl, wr, residual):
    return pl.pallas_call(
        functools.partial(_layer_body, residual=residual),
        grid=(_GRID,),
        in_specs=[
            pl.BlockSpec((_BLK, HH), lambda i: (i, 0)),
            pl.BlockSpec((_BLK, HH), lambda i: (i, 0)),
            pl.BlockSpec((_BLK, 8), lambda i: (i, 0)),
            pl.BlockSpec((_BLK, 8), lambda i: (i, 0)),
            pl.BlockSpec((_BLK, H), lambda i: (i, 0)),
            pl.BlockSpec((H, H), lambda i: (0, 0)),
            pl.BlockSpec((1, H), lambda i: (0, 0)),
            pl.BlockSpec((H, H), lambda i: (0, 0)),
        ],
        out_specs=pl.BlockSpec((_BLK, H), lambda i: (i, 0)),
        out_shape=jax.ShapeDtypeStruct((N, H), jnp.float32),
    )(lo, hi, d0, d1, h, wl, bl, wr)


def _final_body(lo_ref, hi_ref, d0_ref, d1_ref, h_ref, wl_ref, bl_ref,
                wr_ref, wo0_ref, bo0_ref, wo1_ref, bo1_ref, out_ref):
    deg = jnp.maximum(d0_ref[:, 0:1] + d1_ref[:, 0:1], 1.0)
    invd = 1.0 / deg
    lo = lo_ref[...] * invd
    hi = hi_ref[...] * invd
    h = h_ref[...]
    acc = (jnp.dot(lo, wl_ref[0:HH, :], preferred_element_type=jnp.float32)
           + jnp.dot(hi, wl_ref[HH:H, :], preferred_element_type=jnp.float32)
           + jnp.dot(h, wr_ref[...], preferred_element_type=jnp.float32)
           + bl_ref[...])
    h3 = _leaky(acc)
    t = _leaky(jnp.dot(h3, wo0_ref[...],
                       preferred_element_type=jnp.float32) + bo0_ref[...])
    out_ref[...] = jnp.dot(t, wo1_ref[...],
                           preferred_element_type=jnp.float32) + bo1_ref[...]


def _final(lo, hi, d0, d1, h, wl, bl, wr, wo0, bo0, wo1, bo1):
    return pl.pallas_call(
        _final_body,
        grid=(_GRID,),
        in_specs=[
            pl.BlockSpec((_BLK, HH), lambda i: (i, 0)),
            pl.BlockSpec((_BLK, HH), lambda i: (i, 0)),
            pl.BlockSpec((_BLK, 8), lambda i: (i, 0)),
            pl.BlockSpec((_BLK, 8), lambda i: (i, 0)),
            pl.BlockSpec((_BLK, H), lambda i: (i, 0)),
            pl.BlockSpec((H, H), lambda i: (0, 0)),
            pl.BlockSpec((1, H), lambda i: (0, 0)),
            pl.BlockSpec((H, H), lambda i: (0, 0)),
            pl.BlockSpec((H, H), lambda i: (0, 0)),
            pl.BlockSpec((1, H), lambda i: (0, 0)),
            pl.BlockSpec((H, H), lambda i: (0, 0)),
            pl.BlockSpec((1, H), lambda i: (0, 0)),
        ],
        out_specs=pl.BlockSpec((_BLK, H), lambda i: (i, 0)),
        out_shape=jax.ShapeDtypeStruct((N, H), jnp.float32),
    )(lo, hi, d0, d1, h, wl, bl, wr, wo0, bo0, wo1, bo1)


@functools.partial(
    pl.kernel,
    out_type=(jax.ShapeDtypeStruct((N, 8), jnp.float32),
              jax.ShapeDtypeStruct((N, 8), jnp.float32)),
    mesh=_mesh,
    scratch_types=[
        pltpu.VMEM((CHUNK, 8), jnp.float32),
        pltpu.VMEM((G, CHUNK), jnp.int32),
        pltpu.VMEM_SHARED((N, 8), jnp.float32),
    ],
    compiler_params=pltpu.CompilerParams(use_tc_tiling_on_sc=False),
)
def _deg_kernel(dst2_hbm, zeros8_hbm, ones_hbm, out0, out1,
                ones_v, dstg, acc):
    c = lax.axis_index("c")
    s = lax.axis_index("s")
    tid = c * NS + s

    @pl.when(s == 0)
    def _():
        pltpu.sync_copy(zeros8_hbm, acc)

    pltpu.sync_copy(ones_hbm, ones_v)
    plsc.subcore_barrier()

    row_base = tid * (NGRP_D * G)

    def do_group(row0, ng):
        pltpu.sync_copy(dst2_hbm.at[pl.ds(row0, ng)], dstg.at[pl.ds(0, ng)])
        for j in range(ng):
            pltpu.sync_copy(ones_v, acc.at[dstg.at[j]], add=True)

    def body(i, _):
        do_group(row_base + i * G, G)
        return 0

    lax.fori_loop(0, NGRP_D, body, 0)

    @pl.when(tid < ECHUNKS - 32 * NGRP_D * G)
    def _():
        do_group(32 * NGRP_D * G + tid, 1)

    plsc.subcore_barrier()

    @pl.when(s == 0)
    def _():
        @pl.when(c == 0)
        def _():
            pltpu.sync_copy(acc, out0)

        @pl.when(c == 1)
        def _():
            pltpu.sync_copy(acc, out1)


@functools.partial(
    pl.kernel,
    out_type=(jax.ShapeDtypeStruct((N, HH), jnp.float32),
              jax.ShapeDtypeStruct((N, HH), jnp.float32)),
    mesh=_mesh,
    scratch_types=[
        pltpu.VMEM((G, CHUNK), jnp.int32),
        pltpu.VMEM((G, CHUNK), jnp.int32),
        pltpu.VMEM((G, CHUNK), jnp.int32),
        pltpu.VMEM((G, CHUNK, HH), jnp.float32),
        pltpu.VMEM_SHARED((N, HH), jnp.float32),
        pltpu.SemaphoreType.DMA,
        pltpu.SemaphoreType.DMA,
        pltpu.SemaphoreType.DMA,
        pltpu.SemaphoreType.DMA,
        pltpu.SemaphoreType.DMA,
        pltpu.SemaphoreType.DMA,
    ],
    compiler_params=pltpu.CompilerParams(use_tc_tiling_on_sc=False),
)
def _seg_kernel(ht_hbm, src2_hbm, dst2_hbm, zeros32_hbm, out_lo, out_hi,
                srcg, idxg, dstg, rows, acc, sg0, sg1, sg2, sg3, sg4, ss):
    c = lax.axis_index("c")
    s = lax.axis_index("s")
    semg = [sg0, sg1, sg2, sg3, sg4]

    @pl.when(s == 0)
    def _():
        pltpu.sync_copy(zeros32_hbm, acc)

    plsc.subcore_barrier()

    row_base = s * (NGRP_L * G)

    def do_group(row0, ng):
        pltpu.sync_copy(src2_hbm.at[pl.ds(row0, ng)], srcg.at[pl.ds(0, ng)])
        pltpu.sync_copy(dst2_hbm.at[pl.ds(row0, ng)], dstg.at[pl.ds(0, ng)])
        gds = []
        for j in range(ng):
            for k in range(CHUNK // 16):
                sv = srcg[j, pl.ds(k * 16, 16)]
                idxg[j, pl.ds(k * 16, 16)] = sv * 2 + c
            gds.append(pltpu.async_copy(ht_hbm.at[idxg.at[j]], rows.at[j],
                                        semg[j]))
        sds = []
        for j in range(ng):
            gds[j].wait()
            sds.append(pltpu.async_copy(rows.at[j], acc.at[dstg.at[j]], ss,
                                        add=True))
        for d in sds:
            d.wait()

    def body(i, _):
        do_group(row_base + i * G, G)
        return 0

    lax.fori_loop(0, NGRP_L, body, 0)

    @pl.when(s < ECHUNKS - NS * NGRP_L * G)
    def _():
        do_group(NS * NGRP_L * G + s, 1)

    plsc.subcore_barrier()

    @pl.when(s == 0)
    def _():
        @pl.when(c == 0)
        def _():
            pltpu.sync_copy(acc, out_lo)

        @pl.when(c == 1)
        def _():
            pltpu.sync_copy(acc, out_hi)


def kernel(x, edge_index, W_e0, b_e0, W_e1, b_e1, Wl, bl, Wr,
           W_o0, b_o0, W_o1, b_o1):
    src2 = edge_index[0].reshape(ECHUNKS, CHUNK)
    dst2 = edge_index[1].reshape(ECHUNKS, CHUNK)

    zeros8 = jnp.zeros((N, 8), jnp.float32)
    zeros32 = jnp.zeros((N, HH), jnp.float32)
    ones = jnp.ones((CHUNK, 8), jnp.float32)

    h = _embed(x, W_e0, b_e0.reshape(1, H), W_e1, b_e1.reshape(1, H))
    d0, d1 = _deg_kernel(dst2, zeros8, ones)

    for i in range(2):
        lo, hi = _seg_kernel(h.reshape(2 * N, HH), src2, dst2, zeros32)
        h = _layer(lo, hi, d0, d1, h, Wl[i], bl[i].reshape(1, H), Wr[i],
                   residual=(i == 0))

    lo, hi = _seg_kernel(h.reshape(2 * N, HH), src2, dst2, zeros32)
    return _final(lo, hi, d0, d1, h, Wl[2], bl[2].reshape(1, H), Wr[2],
                  W_o0, b_o0.reshape(1, H), W_o1, b_o1.reshape(1, H))


# speedup vs baseline: 7.9404x; 1.0049x over previous
import functools

import jax
import jax.numpy as jnp
from jax import lax
from jax.experimental import pallas as pl
from jax.experimental.pallas import tpu as pltpu
from jax.experimental.pallas import tpu_sc as plsc

N = 50000
E = 800000
H = 64
HH = H // 2
NC = 2
NS = 16
CHUNK = 128

G = 5
GE = G * CHUNK

EPT_L = E // NS
NGRP_L = EPT_L // GE

EPT_D = E // (NC * NS)
NGRP_D = EPT_D // GE

ECHUNKS = E // CHUNK

_mesh = plsc.VectorSubcoreMesh(core_axis_name="c", subcore_axis_name="s")


def _leaky(v):
    return jnp.where(v >= 0, v, 0.01 * v)


_BLK = 2000
_GRID = N // _BLK


def _embed_body(x_ref, we0_ref, be0_ref, we1_ref, be1_ref, olo_ref, ohi_ref):
    h = _leaky(x_ref[...] * we0_ref[...] + be0_ref[...])
    h = jnp.dot(h, we1_ref[...], preferred_element_type=jnp.float32)
    h = _leaky(h + be1_ref[...])
    olo_ref[...] = h[:, 0:HH]
    ohi_ref[...] = h[:, HH:H]


def _embed(x, we0, be0, we1, be1):
    return pl.pallas_call(
        _embed_body,
        grid=(_GRID,),
        in_specs=[
            pl.BlockSpec((_BLK, 1), lambda i: (i, 0)),
            pl.BlockSpec((1, H), lambda i: (0, 0)),
            pl.BlockSpec((1, H), lambda i: (0, 0)),
            pl.BlockSpec((H, H), lambda i: (0, 0)),
            pl.BlockSpec((1, H), lambda i: (0, 0)),
        ],
        out_specs=(pl.BlockSpec((_BLK, HH), lambda i: (i, 0)),
                   pl.BlockSpec((_BLK, HH), lambda i: (i, 0))),
        out_shape=(jax.ShapeDtypeStruct((N, HH), jnp.float32),
                   jax.ShapeDtypeStruct((N, HH), jnp.float32)),
    )(x, we0, be0, we1, be1)


def _layer_body(lo_ref, hi_ref, d0_ref, d1_ref, hlo_ref, hhi_ref, w_ref,
                bl_ref, olo_ref, ohi_ref, *, residual):
    invd = 1.0 / jnp.maximum(d0_ref[:, 0:1] + d1_ref[:, 0:1], 1.0)
    hlo = hlo_ref[...]
    hhi = hhi_ref[...]
    z = jnp.concatenate(
        [lo_ref[...] * invd, hi_ref[...] * invd, hlo, hhi], axis=1)
    act = _leaky(jnp.dot(z, w_ref[...], preferred_element_type=jnp.float32)
                 + bl_ref[...])
    if residual:
        act = act + jnp.concatenate([hlo, hhi], axis=1)
    olo_ref[...] = act[:, 0:HH]
    ohi_ref[...] = act[:, HH:H]


def _layer(lo, hi, d0, d1, hlo, hhi, w, bl, residual):
    return pl.pallas_call(
        functools.partial(_layer_body, residual=residual),
        grid=(_GRID,),
        in_specs=[
            pl.BlockSpec((_BLK, HH), lambda i: (i, 0)),
            pl.BlockSpec((_BLK, HH), lambda i: (i, 0)),
            pl.BlockSpec((_BLK, 8), lambda i: (i, 0)),
            pl.BlockSpec((_BLK, 8), lambda i: (i, 0)),
            pl.BlockSpec((_BLK, HH), lambda i: (i, 0)),
            pl.BlockSpec((_BLK, HH), lambda i: (i, 0)),
            pl.BlockSpec((2 * H, H), lambda i: (0, 0)),
            pl.BlockSpec((1, H), lambda i: (0, 0)),
        ],
        out_specs=(pl.BlockSpec((_BLK, HH), lambda i: (i, 0)),
                   pl.BlockSpec((_BLK, HH), lambda i: (i, 0))),
        out_shape=(jax.ShapeDtypeStruct((N, HH), jnp.float32),
                   jax.ShapeDtypeStruct((N, HH), jnp.float32)),
    )(lo, hi, d0, d1, hlo, hhi, w, bl)


def _final_body(lo_ref, hi_ref, d0_ref, d1_ref, hlo_ref, hhi_ref, w_ref,
                bl_ref, wo0_ref, bo0_ref, wo1_ref, bo1_ref, out_ref):
    invd = 1.0 / jnp.maximum(d0_ref[:, 0:1] + d1_ref[:, 0:1], 1.0)
    z = jnp.concatenate(
        [lo_ref[...] * invd, hi_ref[...] * invd, hlo_ref[...], hhi_ref[...]],
        axis=1)
    h3 = _leaky(jnp.dot(z, w_ref[...], preferred_element_type=jnp.float32)
                + bl_ref[...])
    t = _leaky(jnp.dot(h3, wo0_ref[...],
                       preferred_element_type=jnp.float32) + bo0_ref[...])
    out_ref[...] = jnp.dot(t, wo1_ref[...],
                           preferred_element_type=jnp.float32) + bo1_ref[...]


def _final(lo, hi, d0, d1, hlo, hhi, w, bl, wo0, bo0, wo1, bo1):
    return pl.pallas_call(
        _final_body,
        grid=(_GRID,),
        in_specs=[
            pl.BlockSpec((_BLK, HH), lambda i: (i, 0)),
            pl.BlockSpec((_BLK, HH), lambda i: (i, 0)),
            pl.BlockSpec((_BLK, 8), lambda i: (i, 0)),
            pl.BlockSpec((_BLK, 8), lambda i: (i, 0)),
            pl.BlockSpec((_BLK, HH), lambda i: (i, 0)),
            pl.BlockSpec((_BLK, HH), lambda i: (i, 0)),
            pl.BlockSpec((2 * H, H), lambda i: (0, 0)),
            pl.BlockSpec((1, H), lambda i: (0, 0)),
            pl.BlockSpec((H, H), lambda i: (0, 0)),
            pl.BlockSpec((1, H), lambda i: (0, 0)),
            pl.BlockSpec((H, H), lambda i: (0, 0)),
            pl.BlockSpec((1, H), lambda i: (0, 0)),
        ],
        out_specs=pl.BlockSpec((_BLK, H), lambda i: (i, 0)),
        out_shape=jax.ShapeDtypeStruct((N, H), jnp.float32),
    )(lo, hi, d0, d1, hlo, hhi, w, bl, wo0, bo0, wo1, bo1)


@functools.partial(
    pl.kernel,
    out_type=(jax.ShapeDtypeStruct((N, 8), jnp.float32),
              jax.ShapeDtypeStruct((N, 8), jnp.float32)),
    mesh=_mesh,
    scratch_types=[
        pltpu.VMEM((CHUNK, 8), jnp.float32),
        pltpu.VMEM((G, CHUNK), jnp.int32),
        pltpu.VMEM_SHARED((N, 8), jnp.float32),
    ],
    compiler_params=pltpu.CompilerParams(use_tc_tiling_on_sc=False),
)
def _deg_kernel(ei_hbm, zeros8_hbm, ones_hbm, out0, out1, ones_v, dstg, acc):
    c = lax.axis_index("c")
    s = lax.axis_index("s")
    tid = c * NS + s

    @pl.when(s == 0)
    def _():
        pltpu.sync_copy(zeros8_hbm, acc)

    pltpu.sync_copy(ones_hbm, ones_v)
    plsc.subcore_barrier()

    row_base = ECHUNKS + tid * (NGRP_D * G)

    def do_group(row0, ng):
        pltpu.sync_copy(ei_hbm.at[pl.ds(row0, ng)], dstg.at[pl.ds(0, ng)])
        for j in range(ng):
            pltpu.sync_copy(ones_v, acc.at[dstg.at[j]], add=True)

    def body(i, _):
        do_group(row_base + i * G, G)
        return 0

    lax.fori_loop(0, NGRP_D, body, 0)

    @pl.when(tid < ECHUNKS - 32 * NGRP_D * G)
    def _():
        do_group(ECHUNKS + 32 * NGRP_D * G + tid, 1)

    plsc.subcore_barrier()

    @pl.when(s == 0)
    def _():
        @pl.when(c == 0)
        def _():
            pltpu.sync_copy(acc, out0)

        @pl.when(c == 1)
        def _():
            pltpu.sync_copy(acc, out1)


@functools.partial(
    pl.kernel,
    out_type=(jax.ShapeDtypeStruct((N, HH), jnp.float32),
              jax.ShapeDtypeStruct((N, HH), jnp.float32)),
    mesh=_mesh,
    scratch_types=[
        pltpu.VMEM((G, CHUNK), jnp.int32),
        pltpu.VMEM((G, CHUNK), jnp.int32),
        pltpu.VMEM((G, CHUNK, HH), jnp.float32),
        pltpu.VMEM_SHARED((N, HH), jnp.float32),
        pltpu.SemaphoreType.DMA,
        pltpu.SemaphoreType.DMA,
        pltpu.SemaphoreType.DMA,
        pltpu.SemaphoreType.DMA,
        pltpu.SemaphoreType.DMA,
        pltpu.SemaphoreType.DMA,
    ],
    compiler_params=pltpu.CompilerParams(use_tc_tiling_on_sc=False),
)
def _seg_kernel(hlo_hbm, hhi_hbm, ei_hbm, zeros32_hbm, out_lo, out_hi,
                srcg, dstg, rows, acc, sg0, sg1, sg2, sg3, sg4, ss):
    c = lax.axis_index("c")
    s = lax.axis_index("s")
    semg = [sg0, sg1, sg2, sg3, sg4]

    @pl.when(s == 0)
    def _():
        pltpu.sync_copy(zeros32_hbm, acc)

    plsc.subcore_barrier()

    row_base = s * (NGRP_L * G)

    def run(table):
        def do_group(row0, ng):
            pltpu.sync_copy(ei_hbm.at[pl.ds(row0, ng)],
                            srcg.at[pl.ds(0, ng)])
            pltpu.sync_copy(ei_hbm.at[pl.ds(ECHUNKS + row0, ng)],
                            dstg.at[pl.ds(0, ng)])
            gds = []
            for j in range(ng):
                gds.append(pltpu.async_copy(table.at[srcg.at[j]],
                                            rows.at[j], semg[j]))
            sds = []
            for j in range(ng):
                gds[j].wait()
                sds.append(pltpu.async_copy(rows.at[j], acc.at[dstg.at[j]],
                                            ss, add=True))
            for d in sds:
                d.wait()

        def body(i, _):
            do_group(row_base + i * G, G)
            return 0

        lax.fori_loop(0, NGRP_L, body, 0)

        @pl.when(s < ECHUNKS - NS * NGRP_L * G)
        def _():
            do_group(NS * NGRP_L * G + s, 1)

    @pl.when(c == 0)
    def _():
        run(hlo_hbm)

    @pl.when(c == 1)
    def _():
        run(hhi_hbm)

    plsc.subcore_barrier()

    @pl.when(s == 0)
    def _():
        @pl.when(c == 0)
        def _():
            pltpu.sync_copy(acc, out_lo)

        @pl.when(c == 1)
        def _():
            pltpu.sync_copy(acc, out_hi)


def kernel(x, edge_index, W_e0, b_e0, W_e1, b_e1, Wl, bl, Wr,
           W_o0, b_o0, W_o1, b_o1):
    ei = edge_index.reshape(2 * ECHUNKS, CHUNK)

    zeros8 = jnp.zeros((N, 8), jnp.float32)
    zeros32 = jnp.zeros((N, HH), jnp.float32)
    ones = jnp.ones((CHUNK, 8), jnp.float32)

    hlo, hhi = _embed(x, W_e0, b_e0.reshape(1, H), W_e1, b_e1.reshape(1, H))
    d0, d1 = _deg_kernel(ei, zeros8, ones)

    for i in range(2):
        lo, hi = _seg_kernel(hlo, hhi, ei, zeros32)
        w = jnp.concatenate([Wl[i], Wr[i]], axis=0)
        hlo, hhi = _layer(lo, hi, d0, d1, hlo, hhi, w, bl[i].reshape(1, H),
                          residual=(i == 0))

    lo, hi = _seg_kernel(hlo, hhi, ei, zeros32)
    w = jnp.concatenate([Wl[2], Wr[2]], axis=0)
    return _final(lo, hi, d0, d1, hlo, hhi, w, bl[2].reshape(1, H),
                  W_o0, b_o0.reshape(1, H), W_o1, b_o1.reshape(1, H))


# speedup vs baseline: 8.0887x; 1.0187x over previous
import functools

import jax
import jax.numpy as jnp
from jax import lax
from jax.experimental import pallas as pl
from jax.experimental.pallas import tpu as pltpu
from jax.experimental.pallas import tpu_sc as plsc

N = 50000
E = 800000
H = 64
HH = H // 2
NC = 2
NS = 16
CHUNK = 128

G = 5
GE = G * CHUNK

GS = 3
NGRP_L = 65

EPT_D = E // (NC * NS)
NGRP_D = EPT_D // GE

ECHUNKS = E // CHUNK

_mesh = plsc.VectorSubcoreMesh(core_axis_name="c", subcore_axis_name="s")


def _leaky(v):
    return jnp.where(v >= 0, v, 0.01 * v)


_BLK = 2000
_GRID = N // _BLK


def _embed_body(x_ref, we0_ref, be0_ref, we1_ref, be1_ref, olo_ref, ohi_ref):
    h = _leaky(x_ref[...] * we0_ref[...] + be0_ref[...])
    h = jnp.dot(h, we1_ref[...], preferred_element_type=jnp.float32)
    h = _leaky(h + be1_ref[...])
    olo_ref[...] = h[:, 0:HH]
    ohi_ref[...] = h[:, HH:H]


def _embed(x, we0, be0, we1, be1):
    return pl.pallas_call(
        _embed_body,
        grid=(_GRID,),
        in_specs=[
            pl.BlockSpec((_BLK, 1), lambda i: (i, 0)),
            pl.BlockSpec((1, H), lambda i: (0, 0)),
            pl.BlockSpec((1, H), lambda i: (0, 0)),
            pl.BlockSpec((H, H), lambda i: (0, 0)),
            pl.BlockSpec((1, H), lambda i: (0, 0)),
        ],
        out_specs=(pl.BlockSpec((_BLK, HH), lambda i: (i, 0)),
                   pl.BlockSpec((_BLK, HH), lambda i: (i, 0))),
        out_shape=(jax.ShapeDtypeStruct((N, HH), jnp.float32),
                   jax.ShapeDtypeStruct((N, HH), jnp.float32)),
    )(x, we0, be0, we1, be1)


def _layer_body(lo_ref, hi_ref, d0_ref, d1_ref, hlo_ref, hhi_ref, w_ref,
                bl_ref, olo_ref, ohi_ref, *, residual):
    invd = 1.0 / jnp.maximum(d0_ref[:, 0:1] + d1_ref[:, 0:1], 1.0)
    hlo = hlo_ref[...]
    hhi = hhi_ref[...]
    z = jnp.concatenate(
        [lo_ref[...] * invd, hi_ref[...] * invd, hlo, hhi], axis=1)
    act = _leaky(jnp.dot(z, w_ref[...], preferred_element_type=jnp.float32)
                 + bl_ref[...])
    if residual:
        act = act + jnp.concatenate([hlo, hhi], axis=1)
    olo_ref[...] = act[:, 0:HH]
    ohi_ref[...] = act[:, HH:H]


def _layer(lo, hi, d0, d1, hlo, hhi, w, bl, residual):
    return pl.pallas_call(
        functools.partial(_layer_body, residual=residual),
        grid=(_GRID,),
        in_specs=[
            pl.BlockSpec((_BLK, HH), lambda i: (i, 0)),
            pl.BlockSpec((_BLK, HH), lambda i: (i, 0)),
            pl.BlockSpec((_BLK, 8), lambda i: (i, 0)),
            pl.BlockSpec((_BLK, 8), lambda i: (i, 0)),
            pl.BlockSpec((_BLK, HH), lambda i: (i, 0)),
            pl.BlockSpec((_BLK, HH), lambda i: (i, 0)),
            pl.BlockSpec((2 * H, H), lambda i: (0, 0)),
            pl.BlockSpec((1, H), lambda i: (0, 0)),
        ],
        out_specs=(pl.BlockSpec((_BLK, HH), lambda i: (i, 0)),
                   pl.BlockSpec((_BLK, HH), lambda i: (i, 0))),
        out_shape=(jax.ShapeDtypeStruct((N, HH), jnp.float32),
                   jax.ShapeDtypeStruct((N, HH), jnp.float32)),
    )(lo, hi, d0, d1, hlo, hhi, w, bl)


def _final_body(lo_ref, hi_ref, d0_ref, d1_ref, hlo_ref, hhi_ref, w_ref,
                bl_ref, wo0_ref, bo0_ref, wo1_ref, bo1_ref, out_ref):
    invd = 1.0 / jnp.maximum(d0_ref[:, 0:1] + d1_ref[:, 0:1], 1.0)
    z = jnp.concatenate(
        [lo_ref[...] * invd, hi_ref[...] * invd, hlo_ref[...], hhi_ref[...]],
        axis=1)
    h3 = _leaky(jnp.dot(z, w_ref[...], preferred_element_type=jnp.float32)
                + bl_ref[...])
    t = _leaky(jnp.dot(h3, wo0_ref[...],
                       preferred_element_type=jnp.float32) + bo0_ref[...])
    out_ref[...] = jnp.dot(t, wo1_ref[...],
                           preferred_element_type=jnp.float32) + bo1_ref[...]


def _final(lo, hi, d0, d1, hlo, hhi, w, bl, wo0, bo0, wo1, bo1):
    return pl.pallas_call(
        _final_body,
        grid=(_GRID,),
        in_specs=[
            pl.BlockSpec((_BLK, HH), lambda i: (i, 0)),
            pl.BlockSpec((_BLK, HH), lambda i: (i, 0)),
            pl.BlockSpec((_BLK, 8), lambda i: (i, 0)),
            pl.BlockSpec((_BLK, 8), lambda i: (i, 0)),
            pl.BlockSpec((_BLK, HH), lambda i: (i, 0)),
            pl.BlockSpec((_BLK, HH), lambda i: (i, 0)),
            pl.BlockSpec((2 * H, H), lambda i: (0, 0)),
            pl.BlockSpec((1, H), lambda i: (0, 0)),
            pl.BlockSpec((H, H), lambda i: (0, 0)),
            pl.BlockSpec((1, H), lambda i: (0, 0)),
            pl.BlockSpec((H, H), lambda i: (0, 0)),
            pl.BlockSpec((1, H), lambda i: (0, 0)),
        ],
        out_specs=pl.BlockSpec((_BLK, H), lambda i: (i, 0)),
        out_shape=jax.ShapeDtypeStruct((N, H), jnp.float32),
    )(lo, hi, d0, d1, hlo, hhi, w, bl, wo0, bo0, wo1, bo1)


@functools.partial(
    pl.kernel,
    out_type=(jax.ShapeDtypeStruct((N, 8), jnp.float32),
              jax.ShapeDtypeStruct((N, 8), jnp.float32)),
    mesh=_mesh,
    scratch_types=[
        pltpu.VMEM((CHUNK, 8), jnp.float32),
        pltpu.VMEM((G, CHUNK), jnp.int32),
        pltpu.VMEM_SHARED((N, 8), jnp.float32),
    ],
    compiler_params=pltpu.CompilerParams(use_tc_tiling_on_sc=False),
)
def _deg_kernel(ei_hbm, zeros8_hbm, ones_hbm, out0, out1, ones_v, dstg, acc):
    c = lax.axis_index("c")
    s = lax.axis_index("s")
    tid = c * NS + s

    @pl.when(s == 0)
    def _():
        pltpu.sync_copy(zeros8_hbm, acc)

    pltpu.sync_copy(ones_hbm, ones_v)
    plsc.subcore_barrier()

    row_base = ECHUNKS + tid * (NGRP_D * G)

    def do_group(row0, ng):
        pltpu.sync_copy(ei_hbm.at[pl.ds(row0, ng)], dstg.at[pl.ds(0, ng)])
        for j in range(ng):
            pltpu.sync_copy(ones_v, acc.at[dstg.at[j]], add=True)

    def body(i, _):
        do_group(row_base + i * G, G)
        return 0

    lax.fori_loop(0, NGRP_D, body, 0)

    @pl.when(tid < ECHUNKS - 32 * NGRP_D * G)
    def _():
        do_group(ECHUNKS + 32 * NGRP_D * G + tid, 1)

    plsc.subcore_barrier()

    @pl.when(s == 0)
    def _():
        @pl.when(c == 0)
        def _():
            pltpu.sync_copy(acc, out0)

        @pl.when(c == 1)
        def _():
            pltpu.sync_copy(acc, out1)


@functools.partial(
    pl.kernel,
    out_type=(jax.ShapeDtypeStruct((N, HH), jnp.float32),
              jax.ShapeDtypeStruct((N, HH), jnp.float32)),
    mesh=_mesh,
    scratch_types=[
        pltpu.VMEM((GS, CHUNK), jnp.int32),
        pltpu.VMEM((GS, CHUNK), jnp.int32),
        pltpu.VMEM((GS, CHUNK, HH), jnp.float32),
        pltpu.VMEM((GS, CHUNK), jnp.int32),
        pltpu.VMEM((GS, CHUNK), jnp.int32),
        pltpu.VMEM((GS, CHUNK, HH), jnp.float32),
        pltpu.VMEM_SHARED((N, HH), jnp.float32),
    ] + [pltpu.SemaphoreType.DMA] * (2 * GS + 2),
    compiler_params=pltpu.CompilerParams(use_tc_tiling_on_sc=False),
)
def _seg_kernel(hlo_hbm, hhi_hbm, ei_hbm, zeros32_hbm, out_lo, out_hi,
                srcA, dstA, rowsA, srcB, dstB, rowsB, acc, *sems):
    c = lax.axis_index("c")
    s = lax.axis_index("s")
    semA = sems[0:GS]
    semB = sems[GS:2 * GS]
    ssA = sems[2 * GS]
    ssB = sems[2 * GS + 1]

    @pl.when(s == 0)
    def _():
        pltpu.sync_copy(zeros32_hbm, acc)

    plsc.subcore_barrier()

    row_base = s * (2 * NGRP_L * GS)

    def run(table):
        def load_and_fire(row0, ng, srcg, dstg, rows, gsems):
            pltpu.sync_copy(ei_hbm.at[pl.ds(row0, ng)],
                            srcg.at[pl.ds(0, ng)])
            pltpu.sync_copy(ei_hbm.at[pl.ds(ECHUNKS + row0, ng)],
                            dstg.at[pl.ds(0, ng)])
            return [pltpu.async_copy(table.at[srcg.at[j]], rows.at[j],
                                     gsems[j]) for j in range(ng)]

        def scatter_all(gds, dstg, rows, ssem):
            sds = []
            for j, g in enumerate(gds):
                g.wait()
                sds.append(pltpu.async_copy(rows.at[j], acc.at[dstg.at[j]],
                                            ssem, add=True))
            return sds

        def body(i, _):
            row_a = row_base + (2 * i) * GS
            row_b = row_base + (2 * i + 1) * GS
            ga = load_and_fire(row_a, GS, srcA, dstA, rowsA, semA)
            gb = load_and_fire(row_b, GS, srcB, dstB, rowsB, semB)
            sa = scatter_all(ga, dstA, rowsA, ssA)
            sb = scatter_all(gb, dstB, rowsB, ssB)
            for d in sa + sb:
                d.wait()
            return 0

        lax.fori_loop(0, NGRP_L, body, 0)

        @pl.when(s < ECHUNKS - NS * 2 * NGRP_L * GS)
        def _():
            ga = load_and_fire(NS * 2 * NGRP_L * GS + s, 1,
                               srcA, dstA, rowsA, semA)
            for d in scatter_all(ga, dstA, rowsA, ssA):
                d.wait()

    @pl.when(c == 0)
    def _():
        run(hlo_hbm)

    @pl.when(c == 1)
    def _():
        run(hhi_hbm)

    plsc.subcore_barrier()

    @pl.when(s == 0)
    def _():
        @pl.when(c == 0)
        def _():
            pltpu.sync_copy(acc, out_lo)

        @pl.when(c == 1)
        def _():
            pltpu.sync_copy(acc, out_hi)


def kernel(x, edge_index, W_e0, b_e0, W_e1, b_e1, Wl, bl, Wr,
           W_o0, b_o0, W_o1, b_o1):
    ei = edge_index.reshape(2 * ECHUNKS, CHUNK)

    zeros8 = jnp.zeros((N, 8), jnp.float32)
    zeros32 = jnp.zeros((N, HH), jnp.float32)
    ones = jnp.ones((CHUNK, 8), jnp.float32)

    hlo, hhi = _embed(x, W_e0, b_e0.reshape(1, H), W_e1, b_e1.reshape(1, H))
    d0, d1 = _deg_kernel(ei, zeros8, ones)

    for i in range(2):
        lo, hi = _seg_kernel(hlo, hhi, ei, zeros32)
        w = jnp.concatenate([Wl[i], Wr[i]], axis=0)
        hlo, hhi = _layer(lo, hi, d0, d1, hlo, hhi, w, bl[i].reshape(1, H),
                          residual=(i == 0))

    lo, hi = _seg_kernel(hlo, hhi, ei, zeros32)
    w = jnp.concatenate([Wl[2], Wr[2]], axis=0)
    return _final(lo, hi, d0, d1, hlo, hhi, w, bl[2].reshape(1, H),
                  W_o0, b_o0.reshape(1, H), W_o1, b_o1.reshape(1, H))


# speedup vs baseline: 8.1083x; 1.0024x over previous
import functools

import jax
import jax.numpy as jnp
from jax import lax
from jax.experimental import pallas as pl
from jax.experimental.pallas import tpu as pltpu
from jax.experimental.pallas import tpu_sc as plsc

N = 50000
E = 800000
H = 64
HH = H // 2
NC = 2
NS = 16
CHUNK = 128

G = 5
GE = G * CHUNK

GS = 3
NGRP_L = 65

EPT_D = E // (NC * NS)
NGRP_D = EPT_D // GE

ECHUNKS = E // CHUNK

_mesh = plsc.VectorSubcoreMesh(core_axis_name="c", subcore_axis_name="s")


def _leaky(v):
    return jnp.where(v >= 0, v, 0.01 * v)


_BLK = 2000
_GRID = N // _BLK


def _embed_body(x_ref, we0_ref, be0_ref, we1_ref, be1_ref, olo_ref, ohi_ref):
    h = _leaky(x_ref[...] * we0_ref[...] + be0_ref[...])
    h = jnp.dot(h, we1_ref[...], preferred_element_type=jnp.float32)
    h = _leaky(h + be1_ref[...])
    olo_ref[...] = h[:, 0:HH]
    ohi_ref[...] = h[:, HH:H]


def _embed(x, we0, be0, we1, be1):
    return pl.pallas_call(
        _embed_body,
        grid=(_GRID,),
        in_specs=[
            pl.BlockSpec((_BLK, 1), lambda i: (i, 0)),
            pl.BlockSpec((1, H), lambda i: (0, 0)),
            pl.BlockSpec((1, H), lambda i: (0, 0)),
            pl.BlockSpec((H, H), lambda i: (0, 0)),
            pl.BlockSpec((1, H), lambda i: (0, 0)),
        ],
        out_specs=(pl.BlockSpec((_BLK, HH), lambda i: (i, 0)),
                   pl.BlockSpec((_BLK, HH), lambda i: (i, 0))),
        out_shape=(jax.ShapeDtypeStruct((N, HH), jnp.float32),
                   jax.ShapeDtypeStruct((N, HH), jnp.float32)),
    )(x, we0, be0, we1, be1)


def _layer_body(lo_ref, hi_ref, d0_ref, d1_ref, hlo_ref, hhi_ref, w_ref,
                bl_ref, olo_ref, ohi_ref, *, residual):
    invd = 1.0 / jnp.maximum(d0_ref[:, 0:1] + d1_ref[:, 0:1], 1.0)
    hlo = hlo_ref[...]
    hhi = hhi_ref[...]
    z = jnp.concatenate(
        [lo_ref[...] * invd, hi_ref[...] * invd, hlo, hhi], axis=1)
    act = _leaky(jnp.dot(z, w_ref[...], preferred_element_type=jnp.float32)
                 + bl_ref[...])
    if residual:
        act = act + jnp.concatenate([hlo, hhi], axis=1)
    olo_ref[...] = act[:, 0:HH]
    ohi_ref[...] = act[:, HH:H]


def _layer(lo, hi, d0, d1, hlo, hhi, w, bl, residual):
    return pl.pallas_call(
        functools.partial(_layer_body, residual=residual),
        grid=(_GRID,),
        in_specs=[
            pl.BlockSpec((_BLK, HH), lambda i: (i, 0)),
            pl.BlockSpec((_BLK, HH), lambda i: (i, 0)),
            pl.BlockSpec((_BLK, 8), lambda i: (i, 0)),
            pl.BlockSpec((_BLK, 8), lambda i: (i, 0)),
            pl.BlockSpec((_BLK, HH), lambda i: (i, 0)),
            pl.BlockSpec((_BLK, HH), lambda i: (i, 0)),
            pl.BlockSpec((2 * H, H), lambda i: (0, 0)),
            pl.BlockSpec((1, H), lambda i: (0, 0)),
        ],
        out_specs=(pl.BlockSpec((_BLK, HH), lambda i: (i, 0)),
                   pl.BlockSpec((_BLK, HH), lambda i: (i, 0))),
        out_shape=(jax.ShapeDtypeStruct((N, HH), jnp.float32),
                   jax.ShapeDtypeStruct((N, HH), jnp.float32)),
    )(lo, hi, d0, d1, hlo, hhi, w, bl)


def _final_body(lo_ref, hi_ref, d0_ref, d1_ref, hlo_ref, hhi_ref, w_ref,
                bl_ref, wo0_ref, bo0_ref, wo1_ref, bo1_ref, out_ref):
    invd = 1.0 / jnp.maximum(d0_ref[:, 0:1] + d1_ref[:, 0:1], 1.0)
    z = jnp.concatenate(
        [lo_ref[...] * invd, hi_ref[...] * invd, hlo_ref[...], hhi_ref[...]],
        axis=1)
    h3 = _leaky(jnp.dot(z, w_ref[...], preferred_element_type=jnp.float32)
                + bl_ref[...])
    t = _leaky(jnp.dot(h3, wo0_ref[...],
                       preferred_element_type=jnp.float32) + bo0_ref[...])
    out_ref[...] = jnp.dot(t, wo1_ref[...],
                           preferred_element_type=jnp.float32) + bo1_ref[...]


def _final(lo, hi, d0, d1, hlo, hhi, w, bl, wo0, bo0, wo1, bo1):
    return pl.pallas_call(
        _final_body,
        grid=(_GRID,),
        in_specs=[
            pl.BlockSpec((_BLK, HH), lambda i: (i, 0)),
            pl.BlockSpec((_BLK, HH), lambda i: (i, 0)),
            pl.BlockSpec((_BLK, 8), lambda i: (i, 0)),
            pl.BlockSpec((_BLK, 8), lambda i: (i, 0)),
            pl.BlockSpec((_BLK, HH), lambda i: (i, 0)),
            pl.BlockSpec((_BLK, HH), lambda i: (i, 0)),
            pl.BlockSpec((2 * H, H), lambda i: (0, 0)),
            pl.BlockSpec((1, H), lambda i: (0, 0)),
            pl.BlockSpec((H, H), lambda i: (0, 0)),
            pl.BlockSpec((1, H), lambda i: (0, 0)),
            pl.BlockSpec((H, H), lambda i: (0, 0)),
            pl.BlockSpec((1, H), lambda i: (0, 0)),
        ],
        out_specs=pl.BlockSpec((_BLK, H), lambda i: (i, 0)),
        out_shape=jax.ShapeDtypeStruct((N, H), jnp.float32),
    )(lo, hi, d0, d1, hlo, hhi, w, bl, wo0, bo0, wo1, bo1)


@functools.partial(
    pl.kernel,
    out_type=(jax.ShapeDtypeStruct((N, 8), jnp.float32),
              jax.ShapeDtypeStruct((N, 8), jnp.float32)),
    mesh=_mesh,
    scratch_types=[
        pltpu.VMEM((CHUNK, 8), jnp.float32),
        pltpu.VMEM((G, CHUNK), jnp.int32),
        pltpu.VMEM_SHARED((N, 8), jnp.float32),
    ],
    compiler_params=pltpu.CompilerParams(use_tc_tiling_on_sc=False),
)
def _deg_kernel(ei_hbm, zeros8_hbm, ones_hbm, out0, out1, ones_v, dstg, acc):
    c = lax.axis_index("c")
    s = lax.axis_index("s")
    tid = c * NS + s

    @pl.when(s == 0)
    def _():
        pltpu.sync_copy(zeros8_hbm, acc)

    pltpu.sync_copy(ones_hbm, ones_v)
    plsc.subcore_barrier()

    row_base = ECHUNKS + tid * (NGRP_D * G)

    def do_group(row0, ng):
        pltpu.sync_copy(ei_hbm.at[pl.ds(row0, ng)], dstg.at[pl.ds(0, ng)])
        for j in range(ng):
            pltpu.sync_copy(ones_v, acc.at[dstg.at[j]], add=True)

    def body(i, _):
        do_group(row_base + i * G, G)
        return 0

    lax.fori_loop(0, NGRP_D, body, 0)

    @pl.when(tid < ECHUNKS - 32 * NGRP_D * G)
    def _():
        do_group(ECHUNKS + 32 * NGRP_D * G + tid, 1)

    plsc.subcore_barrier()

    @pl.when(s == 0)
    def _():
        @pl.when(c == 0)
        def _():
            pltpu.sync_copy(acc, out0)

        @pl.when(c == 1)
        def _():
            pltpu.sync_copy(acc, out1)


@functools.partial(
    pl.kernel,
    out_type=(jax.ShapeDtypeStruct((N, HH), jnp.float32),
              jax.ShapeDtypeStruct((N, HH), jnp.float32)),
    mesh=_mesh,
    scratch_types=[
        pltpu.VMEM((GS * CHUNK,), jnp.int32),
        pltpu.VMEM((GS * CHUNK,), jnp.int32),
        pltpu.VMEM((GS * CHUNK, HH), jnp.float32),
        pltpu.VMEM((GS * CHUNK,), jnp.int32),
        pltpu.VMEM((GS * CHUNK,), jnp.int32),
        pltpu.VMEM((GS * CHUNK, HH), jnp.float32),
        pltpu.VMEM_SHARED((N, HH), jnp.float32),
    ] + [pltpu.SemaphoreType.DMA] * (2 * GS + 2),
    compiler_params=pltpu.CompilerParams(use_tc_tiling_on_sc=False),
)
def _seg_kernel(hlo_hbm, hhi_hbm, ei1_hbm, zeros32_hbm, out_lo, out_hi,
                srcA, dstA, rowsA, srcB, dstB, rowsB, acc, *sems):
    c = lax.axis_index("c")
    s = lax.axis_index("s")
    semA = sems[0:GS]
    semB = sems[GS:2 * GS]
    ssA = sems[2 * GS]
    ssB = sems[2 * GS + 1]

    @pl.when(s == 0)
    def _():
        pltpu.sync_copy(zeros32_hbm, acc)

    plsc.subcore_barrier()

    row_base = s * (2 * NGRP_L * GS)

    def run(table):
        def load_and_fire(row0, srcg, dstg, rows, gsem):
            pltpu.sync_copy(ei1_hbm.at[pl.ds(row0 * CHUNK, GS * CHUNK)],
                            srcg)
            pltpu.sync_copy(
                ei1_hbm.at[pl.ds((ECHUNKS + row0) * CHUNK, GS * CHUNK)],
                dstg)
            return pltpu.async_copy(table.at[srcg], rows, gsem)

        def scatter_set(gd, dstg, rows, ssem):
            gd.wait()
            return pltpu.async_copy(rows, acc.at[dstg], ssem, add=True)

        def body(i, _):
            row_a = row_base + (2 * i) * GS
            row_b = row_base + (2 * i + 1) * GS
            ga = load_and_fire(row_a, srcA, dstA, rowsA, semA[0])
            gb = load_and_fire(row_b, srcB, dstB, rowsB, semB[0])
            sa = scatter_set(ga, dstA, rowsA, ssA)
            sb = scatter_set(gb, dstB, rowsB, ssB)
            sa.wait()
            sb.wait()
            return 0

        lax.fori_loop(0, NGRP_L, body, 0)

        @pl.when(s < ECHUNKS - NS * 2 * NGRP_L * GS)
        def _():
            row0 = NS * 2 * NGRP_L * GS + s
            pltpu.sync_copy(ei1_hbm.at[pl.ds(row0 * CHUNK, CHUNK)],
                            srcA.at[pl.ds(0, CHUNK)])
            pltpu.sync_copy(
                ei1_hbm.at[pl.ds((ECHUNKS + row0) * CHUNK, CHUNK)],
                dstA.at[pl.ds(0, CHUNK)])
            g = pltpu.async_copy(table.at[srcA.at[pl.ds(0, CHUNK)]],
                                 rowsA.at[pl.ds(0, CHUNK)], semA[0])
            g.wait()
            pltpu.sync_copy(rowsA.at[pl.ds(0, CHUNK)],
                            acc.at[dstA.at[pl.ds(0, CHUNK)]], add=True)

    @pl.when(c == 0)
    def _():
        run(hlo_hbm)

    @pl.when(c == 1)
    def _():
        run(hhi_hbm)

    plsc.subcore_barrier()

    @pl.when(s == 0)
    def _():
        @pl.when(c == 0)
        def _():
            pltpu.sync_copy(acc, out_lo)

        @pl.when(c == 1)
        def _():
            pltpu.sync_copy(acc, out_hi)


def kernel(x, edge_index, W_e0, b_e0, W_e1, b_e1, Wl, bl, Wr,
           W_o0, b_o0, W_o1, b_o1):
    ei = edge_index.reshape(2 * ECHUNKS, CHUNK)
    ei1 = edge_index.reshape(2 * E)

    zeros8 = jnp.zeros((N, 8), jnp.float32)
    zeros32 = jnp.zeros((N, HH), jnp.float32)
    ones = jnp.ones((CHUNK, 8), jnp.float32)

    hlo, hhi = _embed(x, W_e0, b_e0.reshape(1, H), W_e1, b_e1.reshape(1, H))
    d0, d1 = _deg_kernel(ei, zeros8, ones)

    for i in range(2):
        lo, hi = _seg_kernel(hlo, hhi, ei1, zeros32)
        w = jnp.concatenate([Wl[i], Wr[i]], axis=0)
        hlo, hhi = _layer(lo, hi, d0, d1, hlo, hhi, w, bl[i].reshape(1, H),
                          residual=(i == 0))

    lo, hi = _seg_kernel(hlo, hhi, ei1, zeros32)
    w = jnp.concatenate([Wl[2], Wr[2]], axis=0)
    return _final(lo, hi, d0, d1, hlo, hhi, w, bl[2].reshape(1, H),
                  W_o0, b_o0.reshape(1, H), W_o1, b_o1.reshape(1, H))


# speedup vs baseline: 8.1739x; 1.0081x over previous
import functools

import jax
import jax.numpy as jnp
from jax import lax
from jax.experimental import pallas as pl
from jax.experimental.pallas import tpu as pltpu
from jax.experimental.pallas import tpu_sc as plsc

N = 50000
E = 800000
H = 64
HH = H // 2
NC = 2
NS = 16
CHUNK = 128

G = 5
GE = G * CHUNK

GS = 3
NGRP_L = 65

EPT_D = E // (NC * NS)
NGRP_D = EPT_D // GE

ECHUNKS = E // CHUNK

_mesh = plsc.VectorSubcoreMesh(core_axis_name="c", subcore_axis_name="s")


def _leaky(v):
    return jnp.where(v >= 0, v, 0.01 * v)


_BLK = 5000
_GRID = N // _BLK


def _embed_body(x_ref, we0_ref, be0_ref, we1_ref, be1_ref, olo_ref, ohi_ref):
    h = _leaky(x_ref[...] * we0_ref[...] + be0_ref[...])
    h = jnp.dot(h, we1_ref[...], preferred_element_type=jnp.float32)
    h = _leaky(h + be1_ref[...])
    olo_ref[...] = h[:, 0:HH]
    ohi_ref[...] = h[:, HH:H]


def _embed(x, we0, be0, we1, be1):
    return pl.pallas_call(
        _embed_body,
        grid=(_GRID,),
        in_specs=[
            pl.BlockSpec((_BLK, 1), lambda i: (i, 0)),
            pl.BlockSpec((1, H), lambda i: (0, 0)),
            pl.BlockSpec((1, H), lambda i: (0, 0)),
            pl.BlockSpec((H, H), lambda i: (0, 0)),
            pl.BlockSpec((1, H), lambda i: (0, 0)),
        ],
        out_specs=(pl.BlockSpec((_BLK, HH), lambda i: (i, 0)),
                   pl.BlockSpec((_BLK, HH), lambda i: (i, 0))),
        out_shape=(jax.ShapeDtypeStruct((N, HH), jnp.float32),
                   jax.ShapeDtypeStruct((N, HH), jnp.float32)),
    )(x, we0, be0, we1, be1)


def _layer_body(lo_ref, hi_ref, d0_ref, d1_ref, hlo_ref, hhi_ref, w_ref,
                bl_ref, olo_ref, ohi_ref, *, residual):
    invd = 1.0 / jnp.maximum(d0_ref[:, 0:1] + d1_ref[:, 0:1], 1.0)
    hlo = hlo_ref[...]
    hhi = hhi_ref[...]
    z = jnp.concatenate(
        [lo_ref[...] * invd, hi_ref[...] * invd, hlo, hhi], axis=1)
    act = _leaky(jnp.dot(z, w_ref[...], preferred_element_type=jnp.float32)
                 + bl_ref[...])
    if residual:
        act = act + jnp.concatenate([hlo, hhi], axis=1)
    olo_ref[...] = act[:, 0:HH]
    ohi_ref[...] = act[:, HH:H]


def _layer(lo, hi, d0, d1, hlo, hhi, w, bl, residual):
    return pl.pallas_call(
        functools.partial(_layer_body, residual=residual),
        grid=(_GRID,),
        in_specs=[
            pl.BlockSpec((_BLK, HH), lambda i: (i, 0)),
            pl.BlockSpec((_BLK, HH), lambda i: (i, 0)),
            pl.BlockSpec((_BLK, 8), lambda i: (i, 0)),
            pl.BlockSpec((_BLK, 8), lambda i: (i, 0)),
            pl.BlockSpec((_BLK, HH), lambda i: (i, 0)),
            pl.BlockSpec((_BLK, HH), lambda i: (i, 0)),
            pl.BlockSpec((2 * H, H), lambda i: (0, 0)),
            pl.BlockSpec((1, H), lambda i: (0, 0)),
        ],
        out_specs=(pl.BlockSpec((_BLK, HH), lambda i: (i, 0)),
                   pl.BlockSpec((_BLK, HH), lambda i: (i, 0))),
        out_shape=(jax.ShapeDtypeStruct((N, HH), jnp.float32),
                   jax.ShapeDtypeStruct((N, HH), jnp.float32)),
    )(lo, hi, d0, d1, hlo, hhi, w, bl)


def _final_body(lo_ref, hi_ref, d0_ref, d1_ref, hlo_ref, hhi_ref, w_ref,
                bl_ref, wo0_ref, bo0_ref, wo1_ref, bo1_ref, out_ref):
    invd = 1.0 / jnp.maximum(d0_ref[:, 0:1] + d1_ref[:, 0:1], 1.0)
    z = jnp.concatenate(
        [lo_ref[...] * invd, hi_ref[...] * invd, hlo_ref[...], hhi_ref[...]],
        axis=1)
    h3 = _leaky(jnp.dot(z, w_ref[...], preferred_element_type=jnp.float32)
                + bl_ref[...])
    t = _leaky(jnp.dot(h3, wo0_ref[...],
                       preferred_element_type=jnp.float32) + bo0_ref[...])
    out_ref[...] = jnp.dot(t, wo1_ref[...],
                           preferred_element_type=jnp.float32) + bo1_ref[...]


def _final(lo, hi, d0, d1, hlo, hhi, w, bl, wo0, bo0, wo1, bo1):
    return pl.pallas_call(
        _final_body,
        grid=(_GRID,),
        in_specs=[
            pl.BlockSpec((_BLK, HH), lambda i: (i, 0)),
            pl.BlockSpec((_BLK, HH), lambda i: (i, 0)),
            pl.BlockSpec((_BLK, 8), lambda i: (i, 0)),
            pl.BlockSpec((_BLK, 8), lambda i: (i, 0)),
            pl.BlockSpec((_BLK, HH), lambda i: (i, 0)),
            pl.BlockSpec((_BLK, HH), lambda i: (i, 0)),
            pl.BlockSpec((2 * H, H), lambda i: (0, 0)),
            pl.BlockSpec((1, H), lambda i: (0, 0)),
            pl.BlockSpec((H, H), lambda i: (0, 0)),
            pl.BlockSpec((1, H), lambda i: (0, 0)),
            pl.BlockSpec((H, H), lambda i: (0, 0)),
            pl.BlockSpec((1, H), lambda i: (0, 0)),
        ],
        out_specs=pl.BlockSpec((_BLK, H), lambda i: (i, 0)),
        out_shape=jax.ShapeDtypeStruct((N, H), jnp.float32),
    )(lo, hi, d0, d1, hlo, hhi, w, bl, wo0, bo0, wo1, bo1)


@functools.partial(
    pl.kernel,
    out_type=(jax.ShapeDtypeStruct((N, 8), jnp.float32),
              jax.ShapeDtypeStruct((N, 8), jnp.float32)),
    mesh=_mesh,
    scratch_types=[
        pltpu.VMEM((CHUNK, 8), jnp.float32),
        pltpu.VMEM((G, CHUNK), jnp.int32),
        pltpu.VMEM_SHARED((N, 8), jnp.float32),
    ],
    compiler_params=pltpu.CompilerParams(use_tc_tiling_on_sc=False),
)
def _deg_kernel(ei_hbm, zeros8_hbm, ones_hbm, out0, out1, ones_v, dstg, acc):
    c = lax.axis_index("c")
    s = lax.axis_index("s")
    tid = c * NS + s

    @pl.when(s == 0)
    def _():
        pltpu.sync_copy(zeros8_hbm, acc)

    pltpu.sync_copy(ones_hbm, ones_v)
    plsc.subcore_barrier()

    row_base = ECHUNKS + tid * (NGRP_D * G)

    def do_group(row0, ng):
        pltpu.sync_copy(ei_hbm.at[pl.ds(row0, ng)], dstg.at[pl.ds(0, ng)])
        for j in range(ng):
            pltpu.sync_copy(ones_v, acc.at[dstg.at[j]], add=True)

    def body(i, _):
        do_group(row_base + i * G, G)
        return 0

    lax.fori_loop(0, NGRP_D, body, 0)

    @pl.when(tid < ECHUNKS - 32 * NGRP_D * G)
    def _():
        do_group(ECHUNKS + 32 * NGRP_D * G + tid, 1)

    plsc.subcore_barrier()

    @pl.when(s == 0)
    def _():
        @pl.when(c == 0)
        def _():
            pltpu.sync_copy(acc, out0)

        @pl.when(c == 1)
        def _():
            pltpu.sync_copy(acc, out1)


@functools.partial(
    pl.kernel,
    out_type=(jax.ShapeDtypeStruct((N, HH), jnp.float32),
              jax.ShapeDtypeStruct((N, HH), jnp.float32)),
    mesh=_mesh,
    scratch_types=[
        pltpu.VMEM((GS * CHUNK,), jnp.int32),
        pltpu.VMEM((GS * CHUNK,), jnp.int32),
        pltpu.VMEM((GS * CHUNK, HH), jnp.float32),
        pltpu.VMEM((GS * CHUNK,), jnp.int32),
        pltpu.VMEM((GS * CHUNK,), jnp.int32),
        pltpu.VMEM((GS * CHUNK, HH), jnp.float32),
        pltpu.VMEM_SHARED((N, HH), jnp.float32),
    ] + [pltpu.SemaphoreType.DMA] * (2 * GS + 2),
    compiler_params=pltpu.CompilerParams(use_tc_tiling_on_sc=False),
)
def _seg_kernel(hlo_hbm, hhi_hbm, ei1_hbm, zeros32_hbm, out_lo, out_hi,
                srcA, dstA, rowsA, srcB, dstB, rowsB, acc, *sems):
    c = lax.axis_index("c")
    s = lax.axis_index("s")
    semA = sems[0:GS]
    semB = sems[GS:2 * GS]
    ssA = sems[2 * GS]
    ssB = sems[2 * GS + 1]

    @pl.when(s == 0)
    def _():
        pltpu.sync_copy(zeros32_hbm, acc)

    plsc.subcore_barrier()

    row_base = s * (2 * NGRP_L * GS)

    def run(table):
        def load_and_fire(row0, srcg, dstg, rows, gsem):
            pltpu.sync_copy(ei1_hbm.at[pl.ds(row0 * CHUNK, GS * CHUNK)],
                            srcg)
            pltpu.sync_copy(
                ei1_hbm.at[pl.ds((ECHUNKS + row0) * CHUNK, GS * CHUNK)],
                dstg)
            return pltpu.async_copy(table.at[srcg], rows, gsem)

        def scatter_set(gd, dstg, rows, ssem):
            gd.wait()
            return pltpu.async_copy(rows, acc.at[dstg], ssem, add=True)

        def body(i, _):
            row_a = row_base + (2 * i) * GS
            row_b = row_base + (2 * i + 1) * GS
            ga = load_and_fire(row_a, srcA, dstA, rowsA, semA[0])
            gb = load_and_fire(row_b, srcB, dstB, rowsB, semB[0])
            sa = scatter_set(ga, dstA, rowsA, ssA)
            sb = scatter_set(gb, dstB, rowsB, ssB)
            sa.wait()
            sb.wait()
            return 0

        lax.fori_loop(0, NGRP_L, body, 0)

        @pl.when(s < ECHUNKS - NS * 2 * NGRP_L * GS)
        def _():
            row0 = NS * 2 * NGRP_L * GS + s
            pltpu.sync_copy(ei1_hbm.at[pl.ds(row0 * CHUNK, CHUNK)],
                            srcA.at[pl.ds(0, CHUNK)])
            pltpu.sync_copy(
                ei1_hbm.at[pl.ds((ECHUNKS + row0) * CHUNK, CHUNK)],
                dstA.at[pl.ds(0, CHUNK)])
            g = pltpu.async_copy(table.at[srcA.at[pl.ds(0, CHUNK)]],
                                 rowsA.at[pl.ds(0, CHUNK)], semA[0])
            g.wait()
            pltpu.sync_copy(rowsA.at[pl.ds(0, CHUNK)],
                            acc.at[dstA.at[pl.ds(0, CHUNK)]], add=True)

    @pl.when(c == 0)
    def _():
        run(hlo_hbm)

    @pl.when(c == 1)
    def _():
        run(hhi_hbm)

    plsc.subcore_barrier()

    @pl.when(s == 0)
    def _():
        @pl.when(c == 0)
        def _():
            pltpu.sync_copy(acc, out_lo)

        @pl.when(c == 1)
        def _():
            pltpu.sync_copy(acc, out_hi)


def kernel(x, edge_index, W_e0, b_e0, W_e1, b_e1, Wl, bl, Wr,
           W_o0, b_o0, W_o1, b_o1):
    ei = edge_index.reshape(2 * ECHUNKS, CHUNK)
    ei1 = edge_index.reshape(2 * E)

    zeros8 = jnp.zeros((N, 8), jnp.float32)
    zeros32 = jnp.zeros((N, HH), jnp.float32)
    ones = jnp.ones((CHUNK, 8), jnp.float32)

    hlo, hhi = _embed(x, W_e0, b_e0.reshape(1, H), W_e1, b_e1.reshape(1, H))
    d0, d1 = _deg_kernel(ei, zeros8, ones)

    for i in range(2):
        lo, hi = _seg_kernel(hlo, hhi, ei1, zeros32)
        w = jnp.concatenate([Wl[i], Wr[i]], axis=0)
        hlo, hhi = _layer(lo, hi, d0, d1, hlo, hhi, w, bl[i].reshape(1, H),
                          residual=(i == 0))

    lo, hi = _seg_kernel(hlo, hhi, ei1, zeros32)
    w = jnp.concatenate([Wl[2], Wr[2]], axis=0)
    return _final(lo, hi, d0, d1, hlo, hhi, w, bl[2].reshape(1, H),
                  W_o0, b_o0.reshape(1, H), W_o1, b_o1.reshape(1, H))
